# baseline (speedup 1.0000x reference)
import functools

import jax
import jax.numpy as jnp
from jax import lax
from jax.experimental import pallas as pl
from jax.experimental.pallas import tpu as pltpu
from jax.experimental.pallas import tpu_sc as plsc

N = 10000
E = 80000
D = 256
H = 64
H2 = 32
LAMB = 5.0

NC, NS, L = 2, 16, 16
NW = NC * NS
N16 = 10240
RPS = N16 // NS
CH = 128
W1 = 48
PT2 = 5120
PT6 = 2560
PT6N = 5120

_mesh = plsc.VectorSubcoreMesh(core_axis_name="c", subcore_axis_name="s",
                               num_cores=NC, num_subcores=NS)


def _pad_idx(a, per, pad, padval):
    a = a.reshape(-1, per)
    return jnp.pad(a, ((0, 0), (0, pad - per)), constant_values=padval).reshape(-1)


def _k1_body(x_ref, w_ref, b_ref, t1_ref, xd_ref):
    y = jnp.dot(x_ref[...], w_ref[...], preferred_element_type=jnp.float32)
    ones = jnp.ones((N, 1), jnp.float32)
    zpad = jnp.zeros((N, W1 - 33), jnp.float32)
    t1_ref[0:N, :] = jnp.concatenate([y[:, 0:32], ones, zpad], axis=1)
    t1_ref[N16:N16 + N, :] = jnp.concatenate([y[:, 32:64], ones, zpad], axis=1)
    xd_ref[...] = y[:, 64:128] + b_ref[...]


def _k1(x, w1cat, b1cat):
    return pl.pallas_call(
        _k1_body,
        out_shape=[jax.ShapeDtypeStruct((2 * N16, W1), jnp.float32),
                   jax.ShapeDtypeStruct((N, H), jnp.float32)],
    )(x, w1cat, b1cat)


NB2 = 4


def _make_segsum(width):
    nch = PT2 // CH

    @functools.partial(
        pl.kernel, mesh=_mesh,
        out_type=jax.ShapeDtypeStruct((2 * N16, width), jnp.float32),
        compiler_params=pltpu.CompilerParams(use_tc_tiling_on_sc=False, needs_layout_passes=False),
        scratch_types=[
            pltpu.VMEM((nch, CH), jnp.int32),
            pltpu.VMEM((nch, CH), jnp.int32),
            pltpu.VMEM((NB2, CH, width), jnp.float32),
            pltpu.VMEM_SHARED((N16, width), jnp.float32),
        ] + [pltpu.SemaphoreType.DMA] * (2 * NB2),
    )
    def k(table, srcp, dstp, zeros, out, src_all, dst_all, rows, acc, *sems):
        semg, semw = sems[:NB2], sems[NB2:]
        c = lax.axis_index("c")
        s = lax.axis_index("s")
        r0 = s * RPS
        pltpu.sync_copy(zeros.at[pl.ds(r0, RPS)], acc.at[pl.ds(r0, RPS)])
        wid = c * NS + s
        pltpu.sync_copy(srcp.at[pl.ds(wid * nch, nch)], src_all)
        pltpu.sync_copy(dstp.at[pl.ds(wid * nch, nch)], dst_all)
        plsc.subcore_barrier()

        def fire(b, ch):
            pltpu.async_copy(table.at[src_all.at[ch]], rows.at[b], semg[b])

        for b in range(NB2):
            fire(b, b)

        def body(cq, carry):
            for b in range(NB2):
                ch = cq * NB2 + b
                pltpu.make_async_copy(table.at[src_all.at[ch]],
                                      rows.at[b], semg[b]).wait()
                pltpu.async_copy(rows.at[b], acc.at[dst_all.at[ch]],
                                 semw[b], add=True)
                pltpu.make_async_copy(rows.at[b], acc.at[dst_all.at[ch]],
                                      semw[b]).wait()
                fire(b, jnp.minimum(ch + NB2, nch - 1))
            return carry

        lax.fori_loop(0, nch // NB2, body, 0)
        for b in range(NB2):
            pltpu.make_async_copy(table.at[src_all.at[0]],
                                  rows.at[b], semg[b]).wait()
        plsc.subcore_barrier()
        pltpu.sync_copy(acc.at[pl.ds(r0, RPS)],
                        out.at[pl.ds(c * N16 + r0, RPS)])

    return k


_segsum48 = _make_segsum(W1)
_segsum64 = _make_segsum(H)


def _k3_body(kacc_ref, xd_ref, z1_ref):
    accp = kacc_ref[0:N, 0:32]
    cp = kacc_ref[0:N, 32:33]
    accn = kacc_ref[N16:N16 + N, 0:32]
    cn = kacc_ref[N16:N16 + N, 32:33]
    rp = 1.0 / jnp.maximum(cp, 1.0)
    rn = 1.0 / jnp.maximum(cn, 1.0)
    pre = jnp.concatenate([accp * rp, accn * rn], axis=1) + xd_ref[...]
    z1_ref[...] = jnp.maximum(pre, 0.0)


def _k3(kacc, xd):
    return pl.pallas_call(
        _k3_body,
        out_shape=jax.ShapeDtypeStruct((N, H), jnp.float32),
    )(kacc, xd)


def _k5_body(macc_ref, kacc_ref, z1_ref, wmp_ref, wmn_ref, wz_ref, b2_ref,
             wd_ref, bd_ref, z_ref, ab_ref):
    cp = kacc_ref[0:N, 32:33]
    cn = kacc_ref[N16:N16 + N, 32:33]
    rp = 1.0 / jnp.maximum(cp, 1.0)
    rn = 1.0 / jnp.maximum(cn, 1.0)
    mp = macc_ref[0:N, :] * rp
    mn = macc_ref[N16:N16 + N, :] * rn
    z = (jnp.dot(mp, wmp_ref[...], preferred_element_type=jnp.float32)
         + jnp.dot(mn, wmn_ref[...], preferred_element_type=jnp.float32)
         + jnp.dot(z1_ref[...], wz_ref[...], preferred_element_type=jnp.float32)
         + b2_ref[...])
    z = jnp.maximum(z, 0.0)
    z_ref[...] = z
    ab_ref[...] = jnp.dot(z, wd_ref[...],
                          preferred_element_type=jnp.float32) + bd_ref[...]


def _k5(macc, kacc, z1, wmp, wmn, wz, b2, wd16, bd16):
    return pl.pallas_call(
        _k5_body,
        out_shape=[jax.ShapeDtypeStruct((N, H), jnp.float32),
                   jax.ShapeDtypeStruct((N, 16), jnp.float32)],
    )(macc, kacc, z1, wmp, wmn, wz, b2, wd16, bd16)


S_POS, S_NEG, S_NONE = 0, NW * PT6, 2 * NW * PT6
S_TOT = 2 * NW * PT6 + NW * PT6N


NCH6 = PT6 // CH
NCH6N = PT6N // CH


@functools.partial(
    pl.kernel, mesh=_mesh,
    out_type=[jax.ShapeDtypeStruct((S_TOT,), jnp.float32),
              jax.ShapeDtypeStruct((NW * 128,), jnp.float32)],
    compiler_params=pltpu.CompilerParams(use_tc_tiling_on_sc=False, needs_layout_passes=False),
    scratch_types=[
        pltpu.VMEM((NCH6, CH), jnp.int32),
        pltpu.VMEM((NCH6, CH), jnp.int32),
        pltpu.VMEM((NCH6, CH), jnp.int32),
        pltpu.VMEM((NCH6N, CH), jnp.int32),
        pltpu.VMEM((NCH6N, CH), jnp.int32),
        pltpu.VMEM((2, CH, 16), jnp.float32),
        pltpu.VMEM((2, CH, 16), jnp.float32),
        pltpu.VMEM((2, CH, H), jnp.float32),
        pltpu.VMEM((2, CH, H), jnp.float32),
        pltpu.VMEM((2, CH, H), jnp.float32),
        pltpu.VMEM((NCH6N * CH,), jnp.float32),
        pltpu.VMEM((128,), jnp.float32),
        pltpu.SemaphoreType.DMA,
        pltpu.SemaphoreType.DMA,
    ],
)
def _k6(ab, z, pe0, pe1, ne0, ne1, no0, no1, kpp, knp,
        s_out, p_out, ia, ib, ik, ioa, iob, bufa, bufb, bufi, bufj, bufk,
        sbuf, pbuf, sem0, sem1):
    c = lax.axis_index("c")
    s = lax.axis_index("s")
    wid = c * NS + s
    iota = lax.broadcasted_iota(jnp.int32, (16,), 0)
    sems = (sem0, sem1)

    def nll_task(i0all, i1all, col, nch, lim, s_base):
        def fire(b, ch):
            pltpu.async_copy(ab.at[i0all.at[ch]], bufa.at[b], sems[b])
            pltpu.async_copy(ab.at[i1all.at[ch]], bufb.at[b], sems[b])

        def drain(b):
            pltpu.make_async_copy(ab.at[i0all.at[0]], bufa.at[b], sems[b]).wait()
            pltpu.make_async_copy(ab.at[i1all.at[0]], bufb.at[b], sems[b]).wait()

        for b in range(2):
            fire(b, b)

        def pair(cq, g_acc):
            for b in range(2):
                ch = cq * 2 + b
                drain(b)
                for g in range(CH // 16):
                    rows = iota + g * 16
                    v = []
                    for j in range(3):
                        aj = plsc.load_gather(bufa.at[b], [rows, jnp.full((16,), j, jnp.int32)])
                        bj = plsc.load_gather(bufb.at[b], [rows, jnp.full((16,), j + 4, jnp.int32)])
                        v.append(aj + bj)
                    m = jnp.maximum(jnp.maximum(v[0], v[1]), v[2])
                    sv = (jnp.exp(v[0] - m) + jnp.exp(v[1] - m) + jnp.exp(v[2] - m))
                    gv = m - v[col]
                    je = ch * CH + g * 16 + iota
                    mask = je < lim
                    g_acc = g_acc + jnp.where(mask, gv, 0.0)
                    sbuf[pl.ds(ch * CH + g * 16, 16)] = jnp.where(mask, sv, 1.0)
                fire(b, jnp.minimum(ch + 2, nch - 1))
            return g_acc

        g_acc = lax.fori_loop(0, nch // 2, pair, jnp.zeros((16,), jnp.float32))
        for b in range(2):
            drain(b)
        pltpu.sync_copy(sbuf.at[pl.ds(0, nch * CH)],
                        s_out.at[pl.ds(s_base + wid * nch * CH, nch * CH)])
        return g_acc

    def trip_task(i0all, i1all, ikall, sign):
        def fire(b, ch):
            pltpu.async_copy(z.at[i0all.at[ch]], bufi.at[b], sems[b])
            pltpu.async_copy(z.at[i1all.at[ch]], bufj.at[b], sems[b])
            pltpu.async_copy(z.at[ikall.at[ch]], bufk.at[b], sems[b])

        def drain(b):
            pltpu.make_async_copy(z.at[i0all.at[0]], bufi.at[b], sems[b]).wait()
            pltpu.make_async_copy(z.at[i1all.at[0]], bufj.at[b], sems[b]).wait()
            pltpu.make_async_copy(z.at[ikall.at[0]], bufk.at[b], sems[b]).wait()

        for b in range(2):
            fire(b, b)

        def pair(cq, t_acc):
            for b in range(2):
                ch = cq * 2 + b
                drain(b)

                def group(g, t_in):
                    rows = iota + g * 16
                    dj = jnp.zeros((16,), jnp.float32)
                    dk = jnp.zeros((16,), jnp.float32)
                    for dd in range(H):
                        cols = jnp.full((16,), dd, jnp.int32)
                        zi = plsc.load_gather(bufi.at[b], [rows, cols])
                        zj = plsc.load_gather(bufj.at[b], [rows, cols])
                        zk = plsc.load_gather(bufk.at[b], [rows, cols])
                        t1 = zi - zj
                        t2 = zi - zk
                        dj = dj + t1 * t1
                        dk = dk + t2 * t2
                    out = (dj - dk) if sign > 0 else (dk - dj)
                    out = jnp.maximum(out, 0.0)
                    je = ch * CH + g * 16 + iota
                    return t_in + jnp.where(je < E // NW, out, 0.0)

                t_acc = lax.fori_loop(0, CH // 16, group, t_acc)
                fire(b, jnp.minimum(ch + 2, NCH6 - 1))
            return t_acc

        t_acc = lax.fori_loop(0, NCH6 // 2, pair, jnp.zeros((16,), jnp.float32))
        for b in range(2):
            drain(b)
        return t_acc

    pltpu.sync_copy(pe0.at[pl.ds(wid * NCH6, NCH6)], ia)
    pltpu.sync_copy(pe1.at[pl.ds(wid * NCH6, NCH6)], ib)
    pltpu.sync_copy(kpp.at[pl.ds(wid * NCH6, NCH6)], ik)
    pltpu.sync_copy(no0.at[pl.ds(wid * NCH6N, NCH6N)], ioa)
    pltpu.sync_copy(no1.at[pl.ds(wid * NCH6N, NCH6N)], iob)
    gp = nll_task(ia, ib, 0, NCH6, 2500, S_POS)
    tp = trip_task(ia, ib, ik, +1)
    g0 = nll_task(ioa, iob, 2, NCH6N, 5000, S_NONE)
    pltpu.sync_copy(ne0.at[pl.ds(wid * NCH6, NCH6)], ia)
    pltpu.sync_copy(ne1.at[pl.ds(wid * NCH6, NCH6)], ib)
    pltpu.sync_copy(knp.at[pl.ds(wid * NCH6, NCH6)], ik)
    gn = nll_task(ia, ib, 1, NCH6, 2500, S_NEG)
    tn = trip_task(ia, ib, ik, -1)

    pbuf[pl.ds(0, 16)] = gp
    pbuf[pl.ds(16, 16)] = gn
    pbuf[pl.ds(32, 16)] = g0
    pbuf[pl.ds(48, 16)] = tp
    pbuf[pl.ds(64, 16)] = tn
    zero16 = jnp.zeros((16,), jnp.float32)
    pbuf[pl.ds(80, 16)] = zero16
    pbuf[pl.ds(96, 16)] = zero16
    pbuf[pl.ds(112, 16)] = zero16
    pltpu.sync_copy(pbuf, p_out.at[pl.ds(wid * 128, 128)])


def _k7_body(s_ref, p_ref, out_ref):
    ls = jnp.log(s_ref[...])
    rp = NW * PT6 // 128
    slp = jnp.sum(ls[0:rp])
    sln = jnp.sum(ls[rp:2 * rp])
    sl0 = jnp.sum(ls[2 * rp:])
    q = jnp.sum(p_ref[...], axis=1)
    gp, gn, g0, tp, tn = q[0], q[1], q[2], q[3], q[4]
    fe = jnp.float32(E)
    nll = ((gp + slp) / fe + (gn + sln) / fe + (g0 + sl0) / (2 * fe)) / 3.0
    loss = nll + LAMB * (tp / fe + tn / fe)
    out_ref[0, 0] = loss


def _k7(s_flat, p_flat):
    s2 = s_flat.reshape(S_TOT // 128, 128)
    p2 = p_flat.reshape(NW, 8, 16).transpose(1, 0, 2).reshape(8, NW * 16)
    return pl.pallas_call(
        _k7_body,
        out_shape=jax.ShapeDtypeStruct((1, 1), jnp.float32),
        out_specs=pl.BlockSpec(memory_space=pltpu.SMEM),
    )(s2, p2)


def kernel(positive_edges, negative_edges, target, X,
           Wp1, bp1, Wn1, bn1, Wp2, bp2, Wn2, bn2, Wd, bd,
           none_edges, k_pos, k_neg):
    del target
    pe0, pe1 = positive_edges[0], positive_edges[1]
    ne0, ne1 = negative_edges[0], negative_edges[1]

    w1cat = jnp.concatenate([Wp1[:D], Wn1[:D], Wp1[D:], Wn1[D:]], axis=1)
    b1cat = jnp.concatenate([bp1, bn1]).reshape(1, H)
    zblk = jnp.zeros((H2, H2), jnp.float32)
    wmp = jnp.block([[Wp2[0:H2], zblk], [zblk, Wn2[0:H2]]])
    wmn = jnp.block([[zblk, Wn2[H2:2 * H2]], [Wp2[H2:2 * H2], zblk]])
    wz = jnp.block([[Wp2[2 * H2:3 * H2], zblk], [zblk, Wn2[2 * H2:3 * H2]]])
    b2 = jnp.concatenate([bp2, bn2]).reshape(1, H)
    wd16 = jnp.zeros((H, 16), jnp.float32).at[:, 0:3].set(Wd[:H]).at[:, 4:7].set(Wd[H:])
    bd16 = jnp.zeros((16,), jnp.float32).at[0:3].set(bd).reshape(1, 16)

    per2 = E // NS
    src2 = jnp.concatenate([_pad_idx(pe0, per2, PT2, 0),
                            _pad_idx(ne0, per2, PT2, 0) + N16]).reshape(-1, CH)
    src4 = jnp.concatenate([_pad_idx(pe0, per2, PT2, 0),
                            _pad_idx(ne0, per2, PT2, 0)]).reshape(-1, CH)
    dst24 = jnp.concatenate([_pad_idx(pe1, per2, PT2, N),
                             _pad_idx(ne1, per2, PT2, N)]).reshape(-1, CH)
    per6 = E // NW
    pe0p = _pad_idx(pe0, per6, PT6, 0).reshape(-1, CH)
    pe1p = _pad_idx(pe1, per6, PT6, 0).reshape(-1, CH)
    ne0p = _pad_idx(ne0, per6, PT6, 0).reshape(-1, CH)
    ne1p = _pad_idx(ne1, per6, PT6, 0).reshape(-1, CH)
    no0p = _pad_idx(none_edges[0], 2 * per6, PT6N, 0).reshape(-1, CH)
    no1p = _pad_idx(none_edges[1], 2 * per6, PT6N, 0).reshape(-1, CH)
    kpp = _pad_idx(k_pos, per6, PT6, 0).reshape(-1, CH)
    knp = _pad_idx(k_neg, per6, PT6, 0).reshape(-1, CH)

    z48 = jnp.zeros((N16, W1), jnp.float32)
    z64 = jnp.zeros((N16, H), jnp.float32)

    t1, xd = _k1(X, w1cat, b1cat)
    kacc = _segsum48(t1, src2, dst24, z48)
    z1 = _k3(kacc, xd)
    macc = _segsum64(z1, src4, dst24, z64)
    z, ab = _k5(macc, kacc, z1, wmp, wmn, wz, b2, wd16, bd16)
    s_flat, p_flat = _k6(ab, z, pe0p, pe1p, ne0p, ne1p, no0p, no1p, kpp, knp)
    loss = _k7(s_flat, p_flat)[0, 0]
    return (loss, z)


# speedup vs baseline: 1.6158x; 1.6158x over previous
import functools

import jax
import jax.numpy as jnp
from jax import lax
from jax.experimental import pallas as pl
from jax.experimental.pallas import tpu as pltpu
from jax.experimental.pallas import tpu_sc as plsc

N = 10000
E = 80000
D = 256
H = 64
H2 = 32
LAMB = 5.0

NC, NS, L = 2, 16, 16
NW = NC * NS
N16 = 10240
RPS = N16 // NS
CH = 128
W1 = 48
PT2 = 5120
PT6 = 2560
PT6N = 5120

_mesh = plsc.VectorSubcoreMesh(core_axis_name="c", subcore_axis_name="s",
                               num_cores=NC, num_subcores=NS)


def _pad_idx(a, per, pad, padval):
    a = a.reshape(-1, per)
    return jnp.pad(a, ((0, 0), (0, pad - per)), constant_values=padval).reshape(-1)


def _k1_body(x_ref, w_ref, b_ref, t1_ref, xd_ref):
    y = jnp.dot(x_ref[...], w_ref[...], preferred_element_type=jnp.float32)
    ones = jnp.ones((N, 1), jnp.float32)
    zpad = jnp.zeros((N, W1 - 33), jnp.float32)
    t1_ref[0:N, :] = jnp.concatenate([y[:, 0:32], ones, zpad], axis=1)
    t1_ref[N16:N16 + N, :] = jnp.concatenate([y[:, 32:64], ones, zpad], axis=1)
    xd_ref[...] = y[:, 64:128] + b_ref[...]


def _k1(x, w1cat, b1cat):
    return pl.pallas_call(
        _k1_body,
        out_shape=[jax.ShapeDtypeStruct((2 * N16, W1), jnp.float32),
                   jax.ShapeDtypeStruct((N, H), jnp.float32)],
    )(x, w1cat, b1cat)


GC2 = 128


def _make_segsum(width):

    @functools.partial(
        pl.kernel, mesh=_mesh,
        out_type=jax.ShapeDtypeStruct((2 * N16, width), jnp.float32),
        compiler_params=pltpu.CompilerParams(use_tc_tiling_on_sc=False, needs_layout_passes=False),
        scratch_types=[
            pltpu.VMEM((PT2 // CH, CH), jnp.int32),
            pltpu.VMEM((PT2 // CH, CH), jnp.int32),
            pltpu.VMEM((CH, width), jnp.float32),
            pltpu.VMEM_SHARED((N16, width), jnp.float32),
            pltpu.SemaphoreType.DMA,
            pltpu.SemaphoreType.DMA,
        ],
    )
    def k(table, srcp, dstp, zeros, out, src_all, dst_all, rows, acc,
          semg, semw):
        c = lax.axis_index("c")
        s = lax.axis_index("s")
        r0 = s * RPS
        pltpu.sync_copy(zeros.at[pl.ds(r0, RPS)], acc.at[pl.ds(r0, RPS)])
        wid = c * NS + s
        pltpu.sync_copy(srcp.at[pl.ds(wid * (PT2 // CH), PT2 // CH)], src_all)
        pltpu.sync_copy(dstp.at[pl.ds(wid * (PT2 // CH), PT2 // CH)], dst_all)
        plsc.subcore_barrier()

        def body(ch, carry):
            pltpu.async_copy(table.at[src_all.at[ch]], rows, semg).wait()
            pltpu.sync_copy(rows, acc.at[dst_all.at[ch]], add=True)
            return carry

        lax.fori_loop(0, PT2 // CH, body, 0)
        plsc.subcore_barrier()
        pltpu.sync_copy(acc.at[pl.ds(r0, RPS)],
                        out.at[pl.ds(c * N16 + r0, RPS)])

    return k


_segsum48 = _make_segsum(W1)
_segsum64 = _make_segsum(H)


def _k3_body(kacc_ref, xd_ref, z1_ref):
    accp = kacc_ref[0:N, 0:32]
    cp = kacc_ref[0:N, 32:33]
    accn = kacc_ref[N16:N16 + N, 0:32]
    cn = kacc_ref[N16:N16 + N, 32:33]
    rp = 1.0 / jnp.maximum(cp, 1.0)
    rn = 1.0 / jnp.maximum(cn, 1.0)
    pre = jnp.concatenate([accp * rp, accn * rn], axis=1) + xd_ref[...]
    z1_ref[...] = jnp.maximum(pre, 0.0)


def _k3(kacc, xd):
    return pl.pallas_call(
        _k3_body,
        out_shape=jax.ShapeDtypeStruct((N, H), jnp.float32),
    )(kacc, xd)


def _k5_body(macc_ref, kacc_ref, z1_ref, wmp_ref, wmn_ref, wz_ref, b2_ref,
             wd_ref, bd_ref, z_ref, ab_ref):
    cp = kacc_ref[0:N, 32:33]
    cn = kacc_ref[N16:N16 + N, 32:33]
    rp = 1.0 / jnp.maximum(cp, 1.0)
    rn = 1.0 / jnp.maximum(cn, 1.0)
    mp = macc_ref[0:N, :] * rp
    mn = macc_ref[N16:N16 + N, :] * rn
    z = (jnp.dot(mp, wmp_ref[...], preferred_element_type=jnp.float32)
         + jnp.dot(mn, wmn_ref[...], preferred_element_type=jnp.float32)
         + jnp.dot(z1_ref[...], wz_ref[...], preferred_element_type=jnp.float32)
         + b2_ref[...])
    z = jnp.maximum(z, 0.0)
    z_ref[...] = z
    ab_ref[...] = jnp.dot(z, wd_ref[...],
                          preferred_element_type=jnp.float32) + bd_ref[...]


def _k5(macc, kacc, z1, wmp, wmn, wz, b2, wd16, bd16):
    return pl.pallas_call(
        _k5_body,
        out_shape=[jax.ShapeDtypeStruct((N, H), jnp.float32),
                   jax.ShapeDtypeStruct((N, 16), jnp.float32)],
    )(macc, kacc, z1, wmp, wmn, wz, b2, wd16, bd16)


S_POS, S_NEG, S_NONE = 0, NW * PT6, 2 * NW * PT6
S_TOT = 2 * NW * PT6 + NW * PT6N


GC6 = 128
GT6 = 128


@functools.partial(
    pl.kernel, mesh=_mesh,
    out_type=[jax.ShapeDtypeStruct((S_TOT,), jnp.float32),
              jax.ShapeDtypeStruct((NW * 128,), jnp.float32)],
    compiler_params=pltpu.CompilerParams(use_tc_tiling_on_sc=False, needs_layout_passes=False),
    scratch_types=[
        pltpu.VMEM((PT6 // CH, CH), jnp.int32),
        pltpu.VMEM((PT6 // CH, CH), jnp.int32),
        pltpu.VMEM((PT6 // CH, CH), jnp.int32),
        pltpu.VMEM((PT6N // CH, CH), jnp.int32),
        pltpu.VMEM((PT6N // CH, CH), jnp.int32),
        pltpu.VMEM((GC6, 16), jnp.float32),
        pltpu.VMEM((GC6, 16), jnp.float32),
        pltpu.VMEM((GT6, H), jnp.float32),
        pltpu.VMEM((GT6, H), jnp.float32),
        pltpu.VMEM((GT6, H), jnp.float32),
        pltpu.VMEM((PT6N,), jnp.float32),
        pltpu.VMEM((128,), jnp.float32),
        pltpu.SemaphoreType.DMA,
    ],
)
def _k6(ab, z, pe0, pe1, ne0, ne1, no0, no1, kpp, knp,
        s_out, p_out, ia, ib, ik, ioa, iob, bufa, bufb, bufi, bufj, bufk,
        sbuf, pbuf, sem):
    c = lax.axis_index("c")
    s = lax.axis_index("s")
    wid = c * NS + s
    iota = lax.broadcasted_iota(jnp.int32, (16,), 0)

    def nll_task(i0all, i1all, col, per_pad, lim, s_base):
        nch = per_pad // GC6

        def chunk(ch, g_acc):
            da = pltpu.async_copy(ab.at[i0all.at[ch]], bufa, sem)
            db = pltpu.async_copy(ab.at[i1all.at[ch]], bufb, sem)
            da.wait()
            db.wait()
            for g in range(GC6 // 16):
                rows = iota + g * 16
                v = []
                for j in range(3):
                    aj = plsc.load_gather(bufa, [rows, jnp.full((16,), j, jnp.int32)])
                    bj = plsc.load_gather(bufb, [rows, jnp.full((16,), j + 4, jnp.int32)])
                    v.append(aj + bj)
                m = jnp.maximum(jnp.maximum(v[0], v[1]), v[2])
                sv = (jnp.exp(v[0] - m) + jnp.exp(v[1] - m) + jnp.exp(v[2] - m))
                gv = m - v[col]
                je = ch * GC6 + g * 16 + iota
                mask = je < lim
                g_acc = g_acc + jnp.where(mask, gv, 0.0)
                sbuf[pl.ds(ch * GC6 + g * 16, 16)] = jnp.where(mask, sv, 1.0)
            return g_acc

        g_acc = lax.fori_loop(0, nch, chunk, jnp.zeros((16,), jnp.float32))
        pltpu.sync_copy(sbuf.at[pl.ds(0, per_pad)],
                        s_out.at[pl.ds(s_base + wid * per_pad, per_pad)])
        return g_acc

    def trip_task(i0all, i1all, ikall, sign):
        nch = PT6 // GT6

        def chunk(ch, t_acc):
            di = pltpu.async_copy(z.at[i0all.at[ch]], bufi, sem)
            dj_ = pltpu.async_copy(z.at[i1all.at[ch]], bufj, sem)
            dk_ = pltpu.async_copy(z.at[ikall.at[ch]], bufk, sem)
            di.wait()
            dj_.wait()
            dk_.wait()

            def group(g, t_in):
                rows = iota + g * 16
                dj = jnp.zeros((16,), jnp.float32)
                dk = jnp.zeros((16,), jnp.float32)
                for dd in range(H):
                    cols = jnp.full((16,), dd, jnp.int32)
                    zi = plsc.load_gather(bufi, [rows, cols])
                    zj = plsc.load_gather(bufj, [rows, cols])
                    zk = plsc.load_gather(bufk, [rows, cols])
                    t1 = zi - zj
                    t2 = zi - zk
                    dj = dj + t1 * t1
                    dk = dk + t2 * t2
                out = (dj - dk) if sign > 0 else (dk - dj)
                out = jnp.maximum(out, 0.0)
                je = ch * GT6 + g * 16 + iota
                return t_in + jnp.where(je < E // NW, out, 0.0)

            return lax.fori_loop(0, GT6 // 16, group, t_acc)

        return lax.fori_loop(0, nch, chunk, jnp.zeros((16,), jnp.float32))

    nr, nrn = PT6 // CH, PT6N // CH
    pltpu.sync_copy(pe0.at[pl.ds(wid * nr, nr)], ia)
    pltpu.sync_copy(pe1.at[pl.ds(wid * nr, nr)], ib)
    pltpu.sync_copy(kpp.at[pl.ds(wid * nr, nr)], ik)
    pltpu.sync_copy(no0.at[pl.ds(wid * nrn, nrn)], ioa)
    pltpu.sync_copy(no1.at[pl.ds(wid * nrn, nrn)], iob)
    gp = nll_task(ia, ib, 0, PT6, 2500, S_POS)
    tp = trip_task(ia, ib, ik, +1)
    g0 = nll_task(ioa, iob, 2, PT6N, 5000, S_NONE)
    pltpu.sync_copy(ne0.at[pl.ds(wid * nr, nr)], ia)
    pltpu.sync_copy(ne1.at[pl.ds(wid * nr, nr)], ib)
    pltpu.sync_copy(knp.at[pl.ds(wid * nr, nr)], ik)
    gn = nll_task(ia, ib, 1, PT6, 2500, S_NEG)
    tn = trip_task(ia, ib, ik, -1)

    pbuf[pl.ds(0, 16)] = gp
    pbuf[pl.ds(16, 16)] = gn
    pbuf[pl.ds(32, 16)] = g0
    pbuf[pl.ds(48, 16)] = tp
    pbuf[pl.ds(64, 16)] = tn
    zero16 = jnp.zeros((16,), jnp.float32)
    pbuf[pl.ds(80, 16)] = zero16
    pbuf[pl.ds(96, 16)] = zero16
    pbuf[pl.ds(112, 16)] = zero16
    pltpu.sync_copy(pbuf, p_out.at[pl.ds(wid * 128, 128)])


def _k7_body(s_ref, p_ref, out_ref):
    ls = jnp.log(s_ref[...])
    rp = NW * PT6 // 128
    slp = jnp.sum(ls[0:rp])
    sln = jnp.sum(ls[rp:2 * rp])
    sl0 = jnp.sum(ls[2 * rp:])
    q = jnp.sum(p_ref[...], axis=1)
    gp, gn, g0, tp, tn = q[0], q[1], q[2], q[3], q[4]
    fe = jnp.float32(E)
    nll = ((gp + slp) / fe + (gn + sln) / fe + (g0 + sl0) / (2 * fe)) / 3.0
    loss = nll + LAMB * (tp / fe + tn / fe)
    out_ref[0, 0] = loss


def _k7(s_flat, p_flat):
    s2 = s_flat.reshape(S_TOT // 128, 128)
    p2 = p_flat.reshape(NW, 8, 16).transpose(1, 0, 2).reshape(8, NW * 16)
    return pl.pallas_call(
        _k7_body,
        out_shape=jax.ShapeDtypeStruct((1, 1), jnp.float32),
        out_specs=pl.BlockSpec(memory_space=pltpu.SMEM),
    )(s2, p2)


def kernel(positive_edges, negative_edges, target, X,
           Wp1, bp1, Wn1, bn1, Wp2, bp2, Wn2, bn2, Wd, bd,
           none_edges, k_pos, k_neg):
    del target
    pe0, pe1 = positive_edges[0], positive_edges[1]
    ne0, ne1 = negative_edges[0], negative_edges[1]

    w1cat = jnp.concatenate([Wp1[:D], Wn1[:D], Wp1[D:], Wn1[D:]], axis=1)
    b1cat = jnp.concatenate([bp1, bn1]).reshape(1, H)
    zblk = jnp.zeros((H2, H2), jnp.float32)
    wmp = jnp.block([[Wp2[0:H2], zblk], [zblk, Wn2[0:H2]]])
    wmn = jnp.block([[zblk, Wn2[H2:2 * H2]], [Wp2[H2:2 * H2], zblk]])
    wz = jnp.block([[Wp2[2 * H2:3 * H2], zblk], [zblk, Wn2[2 * H2:3 * H2]]])
    b2 = jnp.concatenate([bp2, bn2]).reshape(1, H)
    wd16 = jnp.zeros((H, 16), jnp.float32).at[:, 0:3].set(Wd[:H]).at[:, 4:7].set(Wd[H:])
    bd16 = jnp.zeros((16,), jnp.float32).at[0:3].set(bd).reshape(1, 16)

    per2 = E // NS
    src2 = jnp.concatenate([_pad_idx(pe0, per2, PT2, 0),
                            _pad_idx(ne0, per2, PT2, 0) + N16]).reshape(-1, CH)
    src4 = jnp.concatenate([_pad_idx(pe0, per2, PT2, 0),
                            _pad_idx(ne0, per2, PT2, 0)]).reshape(-1, CH)
    dst24 = jnp.concatenate([_pad_idx(pe1, per2, PT2, N),
                             _pad_idx(ne1, per2, PT2, N)]).reshape(-1, CH)
    per6 = E // NW
    pe0p = _pad_idx(pe0, per6, PT6, 0).reshape(-1, CH)
    pe1p = _pad_idx(pe1, per6, PT6, 0).reshape(-1, CH)
    ne0p = _pad_idx(ne0, per6, PT6, 0).reshape(-1, CH)
    ne1p = _pad_idx(ne1, per6, PT6, 0).reshape(-1, CH)
    no0p = _pad_idx(none_edges[0], 2 * per6, PT6N, 0).reshape(-1, CH)
    no1p = _pad_idx(none_edges[1], 2 * per6, PT6N, 0).reshape(-1, CH)
    kpp = _pad_idx(k_pos, per6, PT6, 0).reshape(-1, CH)
    knp = _pad_idx(k_neg, per6, PT6, 0).reshape(-1, CH)

    z48 = jnp.zeros((N16, W1), jnp.float32)
    z64 = jnp.zeros((N16, H), jnp.float32)

    t1, xd = _k1(X, w1cat, b1cat)
    kacc = _segsum48(t1, src2, dst24, z48)
    z1 = _k3(kacc, xd)
    macc = _segsum64(z1, src4, dst24, z64)
    z, ab = _k5(macc, kacc, z1, wmp, wmn, wz, b2, wd16, bd16)
    s_flat, p_flat = _k6(ab, z, pe0p, pe1p, ne0p, ne1p, no0p, no1p, kpp, knp)
    loss = _k7(s_flat, p_flat)[0, 0]
    return (loss, z)


# speedup vs baseline: 1.7396x; 1.0766x over previous
import functools

import jax
import jax.numpy as jnp
from jax import lax
from jax.experimental import pallas as pl
from jax.experimental.pallas import tpu as pltpu
from jax.experimental.pallas import tpu_sc as plsc

N = 10000
E = 80000
D = 256
H = 64
H2 = 32
LAMB = 5.0

NC, NS, L = 2, 16, 16
NW = NC * NS
N16 = 10240
RPS = N16 // NS
CH = 128
W1 = 48
PT2 = 5120
PT6 = 2560
PT6N = 5120

_mesh = plsc.VectorSubcoreMesh(core_axis_name="c", subcore_axis_name="s",
                               num_cores=NC, num_subcores=NS)


def _pad_idx(a, per, pad, padval):
    a = a.reshape(-1, per)
    return jnp.pad(a, ((0, 0), (0, pad - per)), constant_values=padval).reshape(-1)


def _k1_body(x_ref, w_ref, b_ref, t1_ref, xd_ref):
    y = jnp.dot(x_ref[...], w_ref[...], preferred_element_type=jnp.float32)
    ones = jnp.ones((N, 1), jnp.float32)
    zpad = jnp.zeros((N, W1 - 33), jnp.float32)
    t1_ref[0:N, :] = jnp.concatenate([y[:, 0:32], ones, zpad], axis=1)
    t1_ref[N16:N16 + N, :] = jnp.concatenate([y[:, 32:64], ones, zpad], axis=1)
    xd_ref[...] = y[:, 64:128] + b_ref[...]


def _k1(x, w1cat, b1cat):
    return pl.pallas_call(
        _k1_body,
        out_shape=[jax.ShapeDtypeStruct((2 * N16, W1), jnp.float32),
                   jax.ShapeDtypeStruct((N, H), jnp.float32)],
    )(x, w1cat, b1cat)


GC2 = 1024


def _make_segsum(width):

    @functools.partial(
        pl.kernel, mesh=_mesh,
        out_type=jax.ShapeDtypeStruct((2 * N16, width), jnp.float32),
        compiler_params=pltpu.CompilerParams(use_tc_tiling_on_sc=False, needs_layout_passes=False),
        scratch_types=[
            pltpu.VMEM((PT2 // GC2, GC2), jnp.int32),
            pltpu.VMEM((PT2 // GC2, GC2), jnp.int32),
            pltpu.VMEM((GC2, width), jnp.float32),
            pltpu.VMEM_SHARED((N16, width), jnp.float32),
            pltpu.SemaphoreType.DMA,
            pltpu.SemaphoreType.DMA,
        ],
    )
    def k(table, srcp, dstp, zeros, out, src_all, dst_all, rows, acc,
          semg, semw):
        c = lax.axis_index("c")
        s = lax.axis_index("s")
        r0 = s * RPS
        pltpu.sync_copy(zeros.at[pl.ds(r0, RPS)], acc.at[pl.ds(r0, RPS)])
        wid = c * NS + s
        nrow = PT2 // GC2
        pltpu.sync_copy(srcp.at[pl.ds(wid * nrow, nrow)], src_all)
        pltpu.sync_copy(dstp.at[pl.ds(wid * nrow, nrow)], dst_all)
        plsc.subcore_barrier()

        def body(ch, carry):
            pltpu.async_copy(table.at[src_all.at[ch]], rows, semg).wait()
            pltpu.sync_copy(rows, acc.at[dst_all.at[ch]], add=True)
            return carry

        lax.fori_loop(0, PT2 // GC2, body, 0)
        plsc.subcore_barrier()
        pltpu.sync_copy(acc.at[pl.ds(r0, RPS)],
                        out.at[pl.ds(c * N16 + r0, RPS)])

    return k


_segsum48 = _make_segsum(W1)
_segsum64 = _make_segsum(H)


def _k3_body(kacc_ref, xd_ref, z1_ref):
    accp = kacc_ref[0:N, 0:32]
    cp = kacc_ref[0:N, 32:33]
    accn = kacc_ref[N16:N16 + N, 0:32]
    cn = kacc_ref[N16:N16 + N, 32:33]
    rp = 1.0 / jnp.maximum(cp, 1.0)
    rn = 1.0 / jnp.maximum(cn, 1.0)
    pre = jnp.concatenate([accp * rp, accn * rn], axis=1) + xd_ref[...]
    z1_ref[...] = jnp.maximum(pre, 0.0)


def _k3(kacc, xd):
    return pl.pallas_call(
        _k3_body,
        out_shape=jax.ShapeDtypeStruct((N, H), jnp.float32),
    )(kacc, xd)


def _k5_body(macc_ref, kacc_ref, z1_ref, wmp_ref, wmn_ref, wz_ref, b2_ref,
             wd_ref, bd_ref, z_ref, ab_ref):
    cp = kacc_ref[0:N, 32:33]
    cn = kacc_ref[N16:N16 + N, 32:33]
    rp = 1.0 / jnp.maximum(cp, 1.0)
    rn = 1.0 / jnp.maximum(cn, 1.0)
    mp = macc_ref[0:N, :] * rp
    mn = macc_ref[N16:N16 + N, :] * rn
    z = (jnp.dot(mp, wmp_ref[...], preferred_element_type=jnp.float32)
         + jnp.dot(mn, wmn_ref[...], preferred_element_type=jnp.float32)
         + jnp.dot(z1_ref[...], wz_ref[...], preferred_element_type=jnp.float32)
         + b2_ref[...])
    z = jnp.maximum(z, 0.0)
    z_ref[...] = z
    ab_ref[...] = jnp.dot(z, wd_ref[...],
                          preferred_element_type=jnp.float32) + bd_ref[...]


def _k5(macc, kacc, z1, wmp, wmn, wz, b2, wd16, bd16):
    return pl.pallas_call(
        _k5_body,
        out_shape=[jax.ShapeDtypeStruct((N, H), jnp.float32),
                   jax.ShapeDtypeStruct((N, 16), jnp.float32)],
    )(macc, kacc, z1, wmp, wmn, wz, b2, wd16, bd16)


S_POS, S_NEG, S_NONE = 0, NW * PT6, 2 * NW * PT6
S_TOT = 2 * NW * PT6 + NW * PT6N


GC6 = 256
GT6 = 256


@functools.partial(
    pl.kernel, mesh=_mesh,
    out_type=[jax.ShapeDtypeStruct((S_TOT,), jnp.float32),
              jax.ShapeDtypeStruct((NW * 128,), jnp.float32)],
    compiler_params=pltpu.CompilerParams(use_tc_tiling_on_sc=False, needs_layout_passes=False),
    scratch_types=[
        pltpu.VMEM((PT6 // GC6, GC6), jnp.int32),
        pltpu.VMEM((PT6 // GC6, GC6), jnp.int32),
        pltpu.VMEM((PT6 // GC6, GC6), jnp.int32),
        pltpu.VMEM((PT6N // GC6, GC6), jnp.int32),
        pltpu.VMEM((PT6N // GC6, GC6), jnp.int32),
        pltpu.VMEM((GC6, 16), jnp.float32),
        pltpu.VMEM((GC6, 16), jnp.float32),
        pltpu.VMEM((GT6, H), jnp.float32),
        pltpu.VMEM((GT6, H), jnp.float32),
        pltpu.VMEM((GT6, H), jnp.float32),
        pltpu.VMEM((PT6N,), jnp.float32),
        pltpu.VMEM((128,), jnp.float32),
        pltpu.SemaphoreType.DMA,
    ],
)
def _k6(ab, z, pe0, pe1, ne0, ne1, no0, no1, kpp, knp,
        s_out, p_out, ia, ib, ik, ioa, iob, bufa, bufb, bufi, bufj, bufk,
        sbuf, pbuf, sem):
    c = lax.axis_index("c")
    s = lax.axis_index("s")
    wid = c * NS + s
    iota = lax.broadcasted_iota(jnp.int32, (16,), 0)

    def nll_task(i0all, i1all, col, per_pad, lim, s_base):
        nch = per_pad // GC6

        def chunk(ch, g_acc):
            da = pltpu.async_copy(ab.at[i0all.at[ch]], bufa, sem)
            db = pltpu.async_copy(ab.at[i1all.at[ch]], bufb, sem)
            da.wait()
            db.wait()
            for g in range(GC6 // 16):
                rows = iota + g * 16
                v = []
                for j in range(3):
                    aj = plsc.load_gather(bufa, [rows, jnp.full((16,), j, jnp.int32)])
                    bj = plsc.load_gather(bufb, [rows, jnp.full((16,), j + 4, jnp.int32)])
                    v.append(aj + bj)
                m = jnp.maximum(jnp.maximum(v[0], v[1]), v[2])
                sv = (jnp.exp(v[0] - m) + jnp.exp(v[1] - m) + jnp.exp(v[2] - m))
                gv = m - v[col]
                je = ch * GC6 + g * 16 + iota
                mask = je < lim
                g_acc = g_acc + jnp.where(mask, gv, 0.0)
                sbuf[pl.ds(ch * GC6 + g * 16, 16)] = jnp.where(mask, sv, 1.0)
            return g_acc

        g_acc = lax.fori_loop(0, nch, chunk, jnp.zeros((16,), jnp.float32))
        pltpu.sync_copy(sbuf.at[pl.ds(0, per_pad)],
                        s_out.at[pl.ds(s_base + wid * per_pad, per_pad)])
        return g_acc

    def trip_task(i0all, i1all, ikall, sign):
        nch = PT6 // GT6

        def chunk(ch, t_acc):
            di = pltpu.async_copy(z.at[i0all.at[ch]], bufi, sem)
            dj_ = pltpu.async_copy(z.at[i1all.at[ch]], bufj, sem)
            dk_ = pltpu.async_copy(z.at[ikall.at[ch]], bufk, sem)
            di.wait()
            dj_.wait()
            dk_.wait()

            def group(g, t_in):
                rows = iota + g * 16
                dj = jnp.zeros((16,), jnp.float32)
                dk = jnp.zeros((16,), jnp.float32)
                for dd in range(H):
                    cols = jnp.full((16,), dd, jnp.int32)
                    zi = plsc.load_gather(bufi, [rows, cols])
                    zj = plsc.load_gather(bufj, [rows, cols])
                    zk = plsc.load_gather(bufk, [rows, cols])
                    t1 = zi - zj
                    t2 = zi - zk
                    dj = dj + t1 * t1
                    dk = dk + t2 * t2
                out = (dj - dk) if sign > 0 else (dk - dj)
                out = jnp.maximum(out, 0.0)
                je = ch * GT6 + g * 16 + iota
                return t_in + jnp.where(je < E // NW, out, 0.0)

            return lax.fori_loop(0, GT6 // 16, group, t_acc)

        return lax.fori_loop(0, nch, chunk, jnp.zeros((16,), jnp.float32))

    nr, nrn = PT6 // GC6, PT6N // GC6
    pltpu.sync_copy(pe0.at[pl.ds(wid * nr, nr)], ia)
    pltpu.sync_copy(pe1.at[pl.ds(wid * nr, nr)], ib)
    pltpu.sync_copy(kpp.at[pl.ds(wid * nr, nr)], ik)
    pltpu.sync_copy(no0.at[pl.ds(wid * nrn, nrn)], ioa)
    pltpu.sync_copy(no1.at[pl.ds(wid * nrn, nrn)], iob)
    gp = nll_task(ia, ib, 0, PT6, 2500, S_POS)
    tp = trip_task(ia, ib, ik, +1)
    g0 = nll_task(ioa, iob, 2, PT6N, 5000, S_NONE)
    pltpu.sync_copy(ne0.at[pl.ds(wid * nr, nr)], ia)
    pltpu.sync_copy(ne1.at[pl.ds(wid * nr, nr)], ib)
    pltpu.sync_copy(knp.at[pl.ds(wid * nr, nr)], ik)
    gn = nll_task(ia, ib, 1, PT6, 2500, S_NEG)
    tn = trip_task(ia, ib, ik, -1)

    pbuf[pl.ds(0, 16)] = gp
    pbuf[pl.ds(16, 16)] = gn
    pbuf[pl.ds(32, 16)] = g0
    pbuf[pl.ds(48, 16)] = tp
    pbuf[pl.ds(64, 16)] = tn
    zero16 = jnp.zeros((16,), jnp.float32)
    pbuf[pl.ds(80, 16)] = zero16
    pbuf[pl.ds(96, 16)] = zero16
    pbuf[pl.ds(112, 16)] = zero16
    pltpu.sync_copy(pbuf, p_out.at[pl.ds(wid * 128, 128)])


def _k7_body(s_ref, p_ref, out_ref):
    ls = jnp.log(s_ref[...])
    rp = NW * PT6 // 128
    slp = jnp.sum(ls[0:rp])
    sln = jnp.sum(ls[rp:2 * rp])
    sl0 = jnp.sum(ls[2 * rp:])
    q = jnp.sum(p_ref[...], axis=1)
    gp, gn, g0, tp, tn = q[0], q[1], q[2], q[3], q[4]
    fe = jnp.float32(E)
    nll = ((gp + slp) / fe + (gn + sln) / fe + (g0 + sl0) / (2 * fe)) / 3.0
    loss = nll + LAMB * (tp / fe + tn / fe)
    out_ref[0, 0] = loss


def _k7(s_flat, p_flat):
    s2 = s_flat.reshape(S_TOT // 128, 128)
    p2 = p_flat.reshape(NW, 8, 16).transpose(1, 0, 2).reshape(8, NW * 16)
    return pl.pallas_call(
        _k7_body,
        out_shape=jax.ShapeDtypeStruct((1, 1), jnp.float32),
        out_specs=pl.BlockSpec(memory_space=pltpu.SMEM),
    )(s2, p2)


def kernel(positive_edges, negative_edges, target, X,
           Wp1, bp1, Wn1, bn1, Wp2, bp2, Wn2, bn2, Wd, bd,
           none_edges, k_pos, k_neg):
    del target
    pe0, pe1 = positive_edges[0], positive_edges[1]
    ne0, ne1 = negative_edges[0], negative_edges[1]

    w1cat = jnp.concatenate([Wp1[:D], Wn1[:D], Wp1[D:], Wn1[D:]], axis=1)
    b1cat = jnp.concatenate([bp1, bn1]).reshape(1, H)
    zblk = jnp.zeros((H2, H2), jnp.float32)
    wmp = jnp.block([[Wp2[0:H2], zblk], [zblk, Wn2[0:H2]]])
    wmn = jnp.block([[zblk, Wn2[H2:2 * H2]], [Wp2[H2:2 * H2], zblk]])
    wz = jnp.block([[Wp2[2 * H2:3 * H2], zblk], [zblk, Wn2[2 * H2:3 * H2]]])
    b2 = jnp.concatenate([bp2, bn2]).reshape(1, H)
    wd16 = jnp.zeros((H, 16), jnp.float32).at[:, 0:3].set(Wd[:H]).at[:, 4:7].set(Wd[H:])
    bd16 = jnp.zeros((16,), jnp.float32).at[0:3].set(bd).reshape(1, 16)

    per2 = E // NS
    src2 = jnp.concatenate([_pad_idx(pe0, per2, PT2, 0),
                            _pad_idx(ne0, per2, PT2, 0) + N16]).reshape(-1, GC2)
    src4 = jnp.concatenate([_pad_idx(pe0, per2, PT2, 0),
                            _pad_idx(ne0, per2, PT2, 0)]).reshape(-1, GC2)
    dst24 = jnp.concatenate([_pad_idx(pe1, per2, PT2, N),
                             _pad_idx(ne1, per2, PT2, N)]).reshape(-1, GC2)
    per6 = E // NW
    pe0p = _pad_idx(pe0, per6, PT6, 0).reshape(-1, GC6)
    pe1p = _pad_idx(pe1, per6, PT6, 0).reshape(-1, GC6)
    ne0p = _pad_idx(ne0, per6, PT6, 0).reshape(-1, GC6)
    ne1p = _pad_idx(ne1, per6, PT6, 0).reshape(-1, GC6)
    no0p = _pad_idx(none_edges[0], 2 * per6, PT6N, 0).reshape(-1, GC6)
    no1p = _pad_idx(none_edges[1], 2 * per6, PT6N, 0).reshape(-1, GC6)
    kpp = _pad_idx(k_pos, per6, PT6, 0).reshape(-1, GC6)
    knp = _pad_idx(k_neg, per6, PT6, 0).reshape(-1, GC6)

    z48 = jnp.zeros((N16, W1), jnp.float32)
    z64 = jnp.zeros((N16, H), jnp.float32)

    t1, xd = _k1(X, w1cat, b1cat)
    kacc = _segsum48(t1, src2, dst24, z48)
    z1 = _k3(kacc, xd)
    macc = _segsum64(z1, src4, dst24, z64)
    z, ab = _k5(macc, kacc, z1, wmp, wmn, wz, b2, wd16, bd16)
    s_flat, p_flat = _k6(ab, z, pe0p, pe1p, ne0p, ne1p, no0p, no1p, kpp, knp)
    loss = _k7(s_flat, p_flat)[0, 0]
    return (loss, z)


# speedup vs baseline: 2.2470x; 1.2917x over previous
import functools

import jax
import jax.numpy as jnp
from jax import lax
from jax.experimental import pallas as pl
from jax.experimental.pallas import tpu as pltpu
from jax.experimental.pallas import tpu_sc as plsc

N = 10000
E = 80000
D = 256
H = 64
H2 = 32
LAMB = 5.0

NC, NS, L = 2, 16, 16
NW = NC * NS
N16 = 10240
RPS = N16 // NS
CH = 128
W1 = 48
PT2 = 5120
PT6 = 2560
PT6N = 5120

_mesh = plsc.VectorSubcoreMesh(core_axis_name="c", subcore_axis_name="s",
                               num_cores=NC, num_subcores=NS)


def _pad_idx(a, per, pad, padval):
    a = a.reshape(-1, per)
    return jnp.pad(a, ((0, 0), (0, pad - per)), constant_values=padval).reshape(-1)


def _k1_body(x_ref, w_ref, b_ref, t1_ref, xd_ref):
    y = jnp.dot(x_ref[...], w_ref[...], preferred_element_type=jnp.float32)
    ones = jnp.ones((N, 1), jnp.float32)
    zpad = jnp.zeros((N, W1 - 33), jnp.float32)
    t1_ref[0:N, :] = jnp.concatenate([y[:, 0:32], ones, zpad], axis=1)
    t1_ref[N16:N16 + N, :] = jnp.concatenate([y[:, 32:64], ones, zpad], axis=1)
    xd_ref[...] = y[:, 64:128] + b_ref[...]


def _k1(x, w1cat, b1cat):
    return pl.pallas_call(
        _k1_body,
        out_shape=[jax.ShapeDtypeStruct((2 * N16, W1), jnp.float32),
                   jax.ShapeDtypeStruct((N, H), jnp.float32)],
    )(x, w1cat, b1cat)


GC2 = 1024


def _make_segsum(width, table_rows, per_core_half, stage):

    @functools.partial(
        pl.kernel, mesh=_mesh,
        out_type=jax.ShapeDtypeStruct((2 * N16, width), jnp.float32),
        compiler_params=pltpu.CompilerParams(use_tc_tiling_on_sc=False, needs_layout_passes=False),
        scratch_types=[
            pltpu.VMEM((PT2 // GC2, GC2), jnp.int32),
            pltpu.VMEM((PT2 // GC2, GC2), jnp.int32),
            pltpu.VMEM((GC2, width), jnp.float32),
            pltpu.VMEM_SHARED((N16 if stage else 8, width), jnp.float32),
            pltpu.VMEM_SHARED((N16, width), jnp.float32),
            pltpu.SemaphoreType.DMA,
            pltpu.SemaphoreType.DMA,
        ],
    )
    def k(table, srcp, dstp, zeros, out, src_all, dst_all, rows, tbl, acc,
          semg, semw):
        c = lax.axis_index("c")
        s = lax.axis_index("s")
        r0 = s * RPS
        pltpu.sync_copy(zeros.at[pl.ds(r0, RPS)], acc.at[pl.ds(r0, RPS)])
        wid = c * NS + s
        if stage:
            tb = c * N16 if per_core_half else 0
            pltpu.sync_copy(table.at[pl.ds(tb + r0, RPS)],
                            tbl.at[pl.ds(r0, RPS)])
        nrow = PT2 // GC2
        pltpu.sync_copy(srcp.at[pl.ds(wid * nrow, nrow)], src_all)
        pltpu.sync_copy(dstp.at[pl.ds(wid * nrow, nrow)], dst_all)
        plsc.subcore_barrier()

        gtbl = tbl if stage else table

        def body(ch, carry):
            pltpu.async_copy(gtbl.at[src_all.at[ch]], rows, semg).wait()
            pltpu.sync_copy(rows, acc.at[dst_all.at[ch]], add=True)
            return carry

        lax.fori_loop(0, PT2 // GC2, body, 0)
        plsc.subcore_barrier()
        pltpu.sync_copy(acc.at[pl.ds(r0, RPS)],
                        out.at[pl.ds(c * N16 + r0, RPS)])

    return k


_segsum48 = _make_segsum(W1, 2 * N16, True, True)
_segsum64 = _make_segsum(H, N16, False, False)


def _k3_body(kacc_ref, xd_ref, z1_ref):
    accp = kacc_ref[0:N, 0:32]
    cp = kacc_ref[0:N, 32:33]
    accn = kacc_ref[N16:N16 + N, 0:32]
    cn = kacc_ref[N16:N16 + N, 32:33]
    rp = 1.0 / jnp.maximum(cp, 1.0)
    rn = 1.0 / jnp.maximum(cn, 1.0)
    pre = jnp.concatenate([accp * rp, accn * rn], axis=1) + xd_ref[...]
    z1_ref[0:N, :] = jnp.maximum(pre, 0.0)


def _k3(kacc, xd):
    return pl.pallas_call(
        _k3_body,
        out_shape=jax.ShapeDtypeStruct((N16, H), jnp.float32),
    )(kacc, xd)


def _k5_body(macc_ref, kacc_ref, z1_ref, wmp_ref, wmn_ref, wz_ref, b2_ref,
             wd_ref, bd_ref, z_ref, ab_ref):
    cp = kacc_ref[0:N, 32:33]
    cn = kacc_ref[N16:N16 + N, 32:33]
    rp = 1.0 / jnp.maximum(cp, 1.0)
    rn = 1.0 / jnp.maximum(cn, 1.0)
    mp = macc_ref[0:N, :] * rp
    mn = macc_ref[N16:N16 + N, :] * rn
    z = (jnp.dot(mp, wmp_ref[...], preferred_element_type=jnp.float32)
         + jnp.dot(mn, wmn_ref[...], preferred_element_type=jnp.float32)
         + jnp.dot(z1_ref[0:N, :], wz_ref[...], preferred_element_type=jnp.float32)
         + b2_ref[...])
    z = jnp.maximum(z, 0.0)
    z_ref[...] = z
    ab_ref[...] = jnp.dot(z, wd_ref[...],
                          preferred_element_type=jnp.float32) + bd_ref[...]


def _k5(macc, kacc, z1, wmp, wmn, wz, b2, wd16, bd16):
    return pl.pallas_call(
        _k5_body,
        out_shape=[jax.ShapeDtypeStruct((N, H), jnp.float32),
                   jax.ShapeDtypeStruct((N, 16), jnp.float32)],
    )(macc, kacc, z1, wmp, wmn, wz, b2, wd16, bd16)


S_POS, S_NEG, S_NONE = 0, NW * PT6, 2 * NW * PT6
S_TOT = 2 * NW * PT6 + NW * PT6N


GC6 = 128
GT6 = 128


@functools.partial(
    pl.kernel, mesh=_mesh,
    out_type=[jax.ShapeDtypeStruct((S_TOT,), jnp.float32),
              jax.ShapeDtypeStruct((NW * 128,), jnp.float32)],
    compiler_params=pltpu.CompilerParams(use_tc_tiling_on_sc=False, needs_layout_passes=False),
    scratch_types=[
        pltpu.VMEM((PT6 // GC6, GC6), jnp.int32),
        pltpu.VMEM((PT6 // GC6, GC6), jnp.int32),
        pltpu.VMEM((PT6 // GC6, GC6), jnp.int32),
        pltpu.VMEM((PT6N // GC6, GC6), jnp.int32),
        pltpu.VMEM((PT6N // GC6, GC6), jnp.int32),
        pltpu.VMEM((GC6, 16), jnp.float32),
        pltpu.VMEM((GC6, 16), jnp.float32),
        pltpu.VMEM((GT6, H), jnp.float32),
        pltpu.VMEM((GT6, H), jnp.float32),
        pltpu.VMEM((GT6, H), jnp.float32),
        pltpu.VMEM((PT6N,), jnp.float32),
        pltpu.VMEM((128,), jnp.float32),
        pltpu.VMEM_SHARED((N, 16), jnp.float32),
        pltpu.VMEM_SHARED((N, H), jnp.float32),
        pltpu.SemaphoreType.DMA,
    ],
)
def _k6(ab, z, pe0, pe1, ne0, ne1, no0, no1, kpp, knp,
        s_out, p_out, ia, ib, ik, ioa, iob, bufa, bufb, bufi, bufj, bufk,
        sbuf, pbuf, absh, zsh, sem):
    c = lax.axis_index("c")
    s = lax.axis_index("s")
    wid = c * NS + s
    iota = lax.broadcasted_iota(jnp.int32, (16,), 0)
    zr = N // NS
    pltpu.sync_copy(ab.at[pl.ds(s * zr, zr)], absh.at[pl.ds(s * zr, zr)])
    pltpu.sync_copy(z.at[pl.ds(s * zr, zr)], zsh.at[pl.ds(s * zr, zr)])
    plsc.subcore_barrier()

    def nll_task(i0all, i1all, col, per_pad, lim, s_base):
        nch = per_pad // GC6

        def chunk(ch, g_acc):
            da = pltpu.async_copy(absh.at[i0all.at[ch]], bufa, sem)
            db = pltpu.async_copy(absh.at[i1all.at[ch]], bufb, sem)
            da.wait()
            db.wait()
            for g in range(GC6 // 16):
                rows = iota + g * 16
                v = []
                for j in range(3):
                    aj = plsc.load_gather(bufa, [rows, jnp.full((16,), j, jnp.int32)])
                    bj = plsc.load_gather(bufb, [rows, jnp.full((16,), j + 4, jnp.int32)])
                    v.append(aj + bj)
                m = jnp.maximum(jnp.maximum(v[0], v[1]), v[2])
                sv = (jnp.exp(v[0] - m) + jnp.exp(v[1] - m) + jnp.exp(v[2] - m))
                gv = m - v[col]
                je = ch * GC6 + g * 16 + iota
                mask = je < lim
                g_acc = g_acc + jnp.where(mask, gv, 0.0)
                sbuf[pl.ds(ch * GC6 + g * 16, 16)] = jnp.where(mask, sv, 1.0)
            return g_acc

        g_acc = lax.fori_loop(0, nch, chunk, jnp.zeros((16,), jnp.float32))
        pltpu.sync_copy(sbuf.at[pl.ds(0, per_pad)],
                        s_out.at[pl.ds(s_base + wid * per_pad, per_pad)])
        return g_acc

    def trip_task(i0all, i1all, ikall, sign):
        nch = PT6 // GT6

        def chunk(ch, t_acc):
            di = pltpu.async_copy(zsh.at[i0all.at[ch]], bufi, sem)
            dj_ = pltpu.async_copy(zsh.at[i1all.at[ch]], bufj, sem)
            dk_ = pltpu.async_copy(zsh.at[ikall.at[ch]], bufk, sem)
            di.wait()
            dj_.wait()
            dk_.wait()

            def group(g, t_in):
                rows = iota + g * 16
                dj = jnp.zeros((16,), jnp.float32)
                dk = jnp.zeros((16,), jnp.float32)
                for dd in range(H):
                    cols = jnp.full((16,), dd, jnp.int32)
                    zi = plsc.load_gather(bufi, [rows, cols])
                    zj = plsc.load_gather(bufj, [rows, cols])
                    zk = plsc.load_gather(bufk, [rows, cols])
                    t1 = zi - zj
                    t2 = zi - zk
                    dj = dj + t1 * t1
                    dk = dk + t2 * t2
                out = (dj - dk) if sign > 0 else (dk - dj)
                out = jnp.maximum(out, 0.0)
                je = ch * GT6 + g * 16 + iota
                return t_in + jnp.where(je < E // NW, out, 0.0)

            return lax.fori_loop(0, GT6 // 16, group, t_acc)

        return lax.fori_loop(0, nch, chunk, jnp.zeros((16,), jnp.float32))

    nr, nrn = PT6 // GC6, PT6N // GC6
    pltpu.sync_copy(pe0.at[pl.ds(wid * nr, nr)], ia)
    pltpu.sync_copy(pe1.at[pl.ds(wid * nr, nr)], ib)
    pltpu.sync_copy(kpp.at[pl.ds(wid * nr, nr)], ik)
    pltpu.sync_copy(no0.at[pl.ds(wid * nrn, nrn)], ioa)
    pltpu.sync_copy(no1.at[pl.ds(wid * nrn, nrn)], iob)
    gp = nll_task(ia, ib, 0, PT6, 2500, S_POS)
    tp = trip_task(ia, ib, ik, +1)
    g0 = nll_task(ioa, iob, 2, PT6N, 5000, S_NONE)
    pltpu.sync_copy(ne0.at[pl.ds(wid * nr, nr)], ia)
    pltpu.sync_copy(ne1.at[pl.ds(wid * nr, nr)], ib)
    pltpu.sync_copy(knp.at[pl.ds(wid * nr, nr)], ik)
    gn = nll_task(ia, ib, 1, PT6, 2500, S_NEG)
    tn = trip_task(ia, ib, ik, -1)

    pbuf[pl.ds(0, 16)] = gp
    pbuf[pl.ds(16, 16)] = gn
    pbuf[pl.ds(32, 16)] = g0
    pbuf[pl.ds(48, 16)] = tp
    pbuf[pl.ds(64, 16)] = tn
    zero16 = jnp.zeros((16,), jnp.float32)
    pbuf[pl.ds(80, 16)] = zero16
    pbuf[pl.ds(96, 16)] = zero16
    pbuf[pl.ds(112, 16)] = zero16
    pltpu.sync_copy(pbuf, p_out.at[pl.ds(wid * 128, 128)])


def _k7_body(s_ref, p_ref, out_ref):
    ls = jnp.log(s_ref[...])
    rp = NW * PT6 // 128
    slp = jnp.sum(ls[0:rp])
    sln = jnp.sum(ls[rp:2 * rp])
    sl0 = jnp.sum(ls[2 * rp:])
    q = jnp.sum(p_ref[...], axis=1)
    gp, gn, g0, tp, tn = q[0], q[1], q[2], q[3], q[4]
    fe = jnp.float32(E)
    nll = ((gp + slp) / fe + (gn + sln) / fe + (g0 + sl0) / (2 * fe)) / 3.0
    loss = nll + LAMB * (tp / fe + tn / fe)
    out_ref[0, 0] = loss


def _k7(s_flat, p_flat):
    s2 = s_flat.reshape(S_TOT // 128, 128)
    p2 = p_flat.reshape(NW, 8, 16).transpose(1, 0, 2).reshape(8, NW * 16)
    return pl.pallas_call(
        _k7_body,
        out_shape=jax.ShapeDtypeStruct((1, 1), jnp.float32),
        out_specs=pl.BlockSpec(memory_space=pltpu.SMEM),
    )(s2, p2)


def kernel(positive_edges, negative_edges, target, X,
           Wp1, bp1, Wn1, bn1, Wp2, bp2, Wn2, bn2, Wd, bd,
           none_edges, k_pos, k_neg):
    del target
    pe0, pe1 = positive_edges[0], positive_edges[1]
    ne0, ne1 = negative_edges[0], negative_edges[1]

    w1cat = jnp.concatenate([Wp1[:D], Wn1[:D], Wp1[D:], Wn1[D:]], axis=1)
    b1cat = jnp.concatenate([bp1, bn1]).reshape(1, H)
    zblk = jnp.zeros((H2, H2), jnp.float32)
    wmp = jnp.block([[Wp2[0:H2], zblk], [zblk, Wn2[0:H2]]])
    wmn = jnp.block([[zblk, Wn2[H2:2 * H2]], [Wp2[H2:2 * H2], zblk]])
    wz = jnp.block([[Wp2[2 * H2:3 * H2], zblk], [zblk, Wn2[2 * H2:3 * H2]]])
    b2 = jnp.concatenate([bp2, bn2]).reshape(1, H)
    wd16 = jnp.zeros((H, 16), jnp.float32).at[:, 0:3].set(Wd[:H]).at[:, 4:7].set(Wd[H:])
    bd16 = jnp.zeros((16,), jnp.float32).at[0:3].set(bd).reshape(1, 16)

    per2 = E // NS
    src4 = jnp.concatenate([_pad_idx(pe0, per2, PT2, 0),
                            _pad_idx(ne0, per2, PT2, 0)]).reshape(-1, GC2)
    dst24 = jnp.concatenate([_pad_idx(pe1, per2, PT2, N),
                             _pad_idx(ne1, per2, PT2, N)]).reshape(-1, GC2)
    per6 = E // NW
    pe0p = _pad_idx(pe0, per6, PT6, 0).reshape(-1, GC6)
    pe1p = _pad_idx(pe1, per6, PT6, 0).reshape(-1, GC6)
    ne0p = _pad_idx(ne0, per6, PT6, 0).reshape(-1, GC6)
    ne1p = _pad_idx(ne1, per6, PT6, 0).reshape(-1, GC6)
    no0p = _pad_idx(none_edges[0], 2 * per6, PT6N, 0).reshape(-1, GC6)
    no1p = _pad_idx(none_edges[1], 2 * per6, PT6N, 0).reshape(-1, GC6)
    kpp = _pad_idx(k_pos, per6, PT6, 0).reshape(-1, GC6)
    knp = _pad_idx(k_neg, per6, PT6, 0).reshape(-1, GC6)

    z48 = jnp.zeros((N16, W1), jnp.float32)
    z64 = jnp.zeros((N16, H), jnp.float32)

    t1, xd = _k1(X, w1cat, b1cat)
    kacc = _segsum48(t1, src4, dst24, z48)
    z1 = _k3(kacc, xd)
    macc = _segsum64(z1, src4, dst24, z64)
    z, ab = _k5(macc, kacc, z1, wmp, wmn, wz, b2, wd16, bd16)
    s_flat, p_flat = _k6(ab, z, pe0p, pe1p, ne0p, ne1p, no0p, no1p, kpp, knp)
    loss = _k7(s_flat, p_flat)[0, 0]
    return (loss, z)


# speedup vs baseline: 2.3893x; 1.0633x over previous
import functools

import jax
import jax.numpy as jnp
from jax import lax
from jax.experimental import pallas as pl
from jax.experimental.pallas import tpu as pltpu
from jax.experimental.pallas import tpu_sc as plsc

N = 10000
E = 80000
D = 256
H = 64
H2 = 32
LAMB = 5.0

NC, NS, L = 2, 16, 16
NW = NC * NS
N16 = 10240
RPS = N16 // NS
CH = 128
W1 = 48
PT2 = 5120
PT6 = 2560
PT6N = 5120

_mesh = plsc.VectorSubcoreMesh(core_axis_name="c", subcore_axis_name="s",
                               num_cores=NC, num_subcores=NS)


def _pad_idx(a, per, pad, padval):
    a = a.reshape(-1, per)
    return jnp.pad(a, ((0, 0), (0, pad - per)), constant_values=padval).reshape(-1)


def _k1_body(x_ref, w_ref, b_ref, t1_ref, xd_ref):
    y = jnp.dot(x_ref[...], w_ref[...], preferred_element_type=jnp.float32)
    ones = jnp.ones((N, 1), jnp.float32)
    zpad = jnp.zeros((N, W1 - 33), jnp.float32)
    t1_ref[0:N, :] = jnp.concatenate([y[:, 0:32], ones, zpad], axis=1)
    t1_ref[N16:N16 + N, :] = jnp.concatenate([y[:, 32:64], ones, zpad], axis=1)
    xd_ref[...] = y[:, 64:128] + b_ref[...]


def _k1(x, w1cat, b1cat):
    return pl.pallas_call(
        _k1_body,
        out_shape=[jax.ShapeDtypeStruct((2 * N16, W1), jnp.float32),
                   jax.ShapeDtypeStruct((N, H), jnp.float32)],
    )(x, w1cat, b1cat)


GC2 = 1024


def _make_segsum(width, table_rows, per_core_half, stage):

    @functools.partial(
        pl.kernel, mesh=_mesh,
        out_type=jax.ShapeDtypeStruct((2 * N16, width), jnp.float32),
        compiler_params=pltpu.CompilerParams(use_tc_tiling_on_sc=False, needs_layout_passes=False),
        scratch_types=[
            pltpu.VMEM((PT2 // GC2, GC2), jnp.int32),
            pltpu.VMEM((PT2 // GC2, GC2), jnp.int32),
            pltpu.VMEM((GC2, width), jnp.float32),
            pltpu.VMEM_SHARED((N16 if stage else 8, width), jnp.float32),
            pltpu.VMEM_SHARED((N16, width), jnp.float32),
            pltpu.SemaphoreType.DMA,
            pltpu.SemaphoreType.DMA,
        ],
    )
    def k(table, srcp, dstp, zeros, out, src_all, dst_all, rows, tbl, acc,
          semg, semw):
        c = lax.axis_index("c")
        s = lax.axis_index("s")
        r0 = s * RPS
        pltpu.sync_copy(zeros.at[pl.ds(r0, RPS)], acc.at[pl.ds(r0, RPS)])
        wid = c * NS + s
        if stage:
            tb = c * N16 if per_core_half else 0
            pltpu.sync_copy(table.at[pl.ds(tb + r0, RPS)],
                            tbl.at[pl.ds(r0, RPS)])
        nrow = PT2 // GC2
        pltpu.sync_copy(srcp.at[pl.ds(wid * nrow, nrow)], src_all)
        pltpu.sync_copy(dstp.at[pl.ds(wid * nrow, nrow)], dst_all)
        plsc.subcore_barrier()

        gtbl = tbl if stage else table

        def body(ch, carry):
            pltpu.async_copy(gtbl.at[src_all.at[ch]], rows, semg).wait()
            pltpu.sync_copy(rows, acc.at[dst_all.at[ch]], add=True)
            return carry

        lax.fori_loop(0, PT2 // GC2, body, 0)
        plsc.subcore_barrier()
        pltpu.sync_copy(acc.at[pl.ds(r0, RPS)],
                        out.at[pl.ds(c * N16 + r0, RPS)])

    return k


_segsum48 = _make_segsum(W1, 2 * N16, True, True)


@functools.partial(
    pl.kernel, mesh=_mesh,
    out_type=jax.ShapeDtypeStruct((2 * N16, H), jnp.float32),
    compiler_params=pltpu.CompilerParams(use_tc_tiling_on_sc=False, needs_layout_passes=False),
    scratch_types=[
        pltpu.VMEM((PT2 // GC2, GC2), jnp.int32),
        pltpu.VMEM((PT2 // GC2, GC2), jnp.int32),
        pltpu.VMEM((PT2 // GC2, GC2), jnp.int32),
        pltpu.VMEM((PT2 // GC2, GC2), jnp.int32),
        pltpu.VMEM((GC2, H2), jnp.float32),
        pltpu.VMEM_SHARED((N16, H2), jnp.float32),
        pltpu.VMEM_SHARED((N16, H2), jnp.float32),
        pltpu.VMEM_SHARED((N16, H2), jnp.float32),
        pltpu.SemaphoreType.DMA,
    ],
)
def _segsum64(table, srcp, dstp, zeros, out, srca, dsta, srcb, dstb,
              rows, tbl, accp, accn, semg):
    c = lax.axis_index("c")
    s = lax.axis_index("s")
    r0 = s * RPS
    pltpu.sync_copy(zeros.at[pl.ds(r0, RPS), pl.ds(0, H2)],
                    accp.at[pl.ds(r0, RPS)])
    pltpu.sync_copy(zeros.at[pl.ds(r0, RPS), pl.ds(0, H2)],
                    accn.at[pl.ds(r0, RPS)])
    pltpu.sync_copy(table.at[pl.ds(r0, RPS), pl.ds(c * H2, H2)],
                    tbl.at[pl.ds(r0, RPS)])
    nrow = PT2 // GC2
    pltpu.sync_copy(srcp.at[pl.ds(s * nrow, nrow)], srca)
    pltpu.sync_copy(dstp.at[pl.ds(s * nrow, nrow)], dsta)
    pltpu.sync_copy(srcp.at[pl.ds((NS + s) * nrow, nrow)], srcb)
    pltpu.sync_copy(dstp.at[pl.ds((NS + s) * nrow, nrow)], dstb)
    plsc.subcore_barrier()

    def body_p(ch, carry):
        pltpu.async_copy(tbl.at[srca.at[ch]], rows, semg).wait()
        pltpu.sync_copy(rows, accp.at[dsta.at[ch]], add=True)
        return carry

    def body_n(ch, carry):
        pltpu.async_copy(tbl.at[srcb.at[ch]], rows, semg).wait()
        pltpu.sync_copy(rows, accn.at[dstb.at[ch]], add=True)
        return carry

    lax.fori_loop(0, PT2 // GC2, body_p, 0)
    lax.fori_loop(0, PT2 // GC2, body_n, 0)
    plsc.subcore_barrier()
    pltpu.sync_copy(accp.at[pl.ds(r0, RPS)],
                    out.at[pl.ds(r0, RPS), pl.ds(c * H2, H2)])
    pltpu.sync_copy(accn.at[pl.ds(r0, RPS)],
                    out.at[pl.ds(N16 + r0, RPS), pl.ds(c * H2, H2)])


def _k3_body(kacc_ref, xd_ref, z1_ref):
    accp = kacc_ref[0:N, 0:32]
    cp = kacc_ref[0:N, 32:33]
    accn = kacc_ref[N16:N16 + N, 0:32]
    cn = kacc_ref[N16:N16 + N, 32:33]
    rp = 1.0 / jnp.maximum(cp, 1.0)
    rn = 1.0 / jnp.maximum(cn, 1.0)
    pre = jnp.concatenate([accp * rp, accn * rn], axis=1) + xd_ref[...]
    z1_ref[0:N, :] = jnp.maximum(pre, 0.0)


def _k3(kacc, xd):
    return pl.pallas_call(
        _k3_body,
        out_shape=jax.ShapeDtypeStruct((N16, H), jnp.float32),
    )(kacc, xd)


def _k5_body(macc_ref, kacc_ref, z1_ref, wmp_ref, wmn_ref, wz_ref, b2_ref,
             wd_ref, bd_ref, z_ref, ab_ref):
    cp = kacc_ref[0:N, 32:33]
    cn = kacc_ref[N16:N16 + N, 32:33]
    rp = 1.0 / jnp.maximum(cp, 1.0)
    rn = 1.0 / jnp.maximum(cn, 1.0)
    mp = macc_ref[0:N, :] * rp
    mn = macc_ref[N16:N16 + N, :] * rn
    z = (jnp.dot(mp, wmp_ref[...], preferred_element_type=jnp.float32)
         + jnp.dot(mn, wmn_ref[...], preferred_element_type=jnp.float32)
         + jnp.dot(z1_ref[0:N, :], wz_ref[...], preferred_element_type=jnp.float32)
         + b2_ref[...])
    z = jnp.maximum(z, 0.0)
    z_ref[...] = z
    ab_ref[...] = jnp.dot(z, wd_ref[...],
                          preferred_element_type=jnp.float32) + bd_ref[...]


def _k5(macc, kacc, z1, wmp, wmn, wz, b2, wd16, bd16):
    return pl.pallas_call(
        _k5_body,
        out_shape=[jax.ShapeDtypeStruct((N, H), jnp.float32),
                   jax.ShapeDtypeStruct((N, 16), jnp.float32)],
    )(macc, kacc, z1, wmp, wmn, wz, b2, wd16, bd16)


S_POS, S_NEG, S_NONE = 0, NW * PT6, 2 * NW * PT6
S_TOT = 2 * NW * PT6 + NW * PT6N


GC6 = 128
GT6 = 128


@functools.partial(
    pl.kernel, mesh=_mesh,
    out_type=[jax.ShapeDtypeStruct((S_TOT,), jnp.float32),
              jax.ShapeDtypeStruct((NW * 128,), jnp.float32)],
    compiler_params=pltpu.CompilerParams(use_tc_tiling_on_sc=False, needs_layout_passes=False),
    scratch_types=[
        pltpu.VMEM((PT6 // GC6, GC6), jnp.int32),
        pltpu.VMEM((PT6 // GC6, GC6), jnp.int32),
        pltpu.VMEM((PT6 // GC6, GC6), jnp.int32),
        pltpu.VMEM((PT6N // GC6, GC6), jnp.int32),
        pltpu.VMEM((PT6N // GC6, GC6), jnp.int32),
        pltpu.VMEM((GC6, 16), jnp.float32),
        pltpu.VMEM((GC6, 16), jnp.float32),
        pltpu.VMEM((GT6, H), jnp.float32),
        pltpu.VMEM((GT6, H), jnp.float32),
        pltpu.VMEM((GT6, H), jnp.float32),
        pltpu.VMEM((PT6N,), jnp.float32),
        pltpu.VMEM((128,), jnp.float32),
        pltpu.VMEM_SHARED((N, 16), jnp.float32),
        pltpu.VMEM_SHARED((N, H), jnp.float32),
        pltpu.SemaphoreType.DMA,
    ],
)
def _k6(ab, z, pe0, pe1, ne0, ne1, no0, no1, kpp, knp,
        s_out, p_out, ia, ib, ik, ioa, iob, bufa, bufb, bufi, bufj, bufk,
        sbuf, pbuf, absh, zsh, sem):
    c = lax.axis_index("c")
    s = lax.axis_index("s")
    wid = c * NS + s
    iota = lax.broadcasted_iota(jnp.int32, (16,), 0)
    zr = N // NS
    pltpu.sync_copy(ab.at[pl.ds(s * zr, zr)], absh.at[pl.ds(s * zr, zr)])
    pltpu.sync_copy(z.at[pl.ds(s * zr, zr)], zsh.at[pl.ds(s * zr, zr)])
    plsc.subcore_barrier()

    def nll_task(i0all, i1all, col, per_pad, lim, s_base):
        nch = per_pad // GC6

        def chunk(ch, g_acc):
            da = pltpu.async_copy(absh.at[i0all.at[ch]], bufa, sem)
            db = pltpu.async_copy(absh.at[i1all.at[ch]], bufb, sem)
            da.wait()
            db.wait()
            for g in range(GC6 // 16):
                rows = iota + g * 16
                v = []
                for j in range(3):
                    aj = plsc.load_gather(bufa, [rows, jnp.full((16,), j, jnp.int32)])
                    bj = plsc.load_gather(bufb, [rows, jnp.full((16,), j + 4, jnp.int32)])
                    v.append(aj + bj)
                m = jnp.maximum(jnp.maximum(v[0], v[1]), v[2])
                sv = (jnp.exp(v[0] - m) + jnp.exp(v[1] - m) + jnp.exp(v[2] - m))
                gv = m - v[col]
                je = ch * GC6 + g * 16 + iota
                mask = je < lim
                g_acc = g_acc + jnp.where(mask, gv, 0.0)
                sbuf[pl.ds(ch * GC6 + g * 16, 16)] = jnp.where(mask, sv, 1.0)
            return g_acc

        g_acc = lax.fori_loop(0, nch, chunk, jnp.zeros((16,), jnp.float32))
        pltpu.sync_copy(sbuf.at[pl.ds(0, per_pad)],
                        s_out.at[pl.ds(s_base + wid * per_pad, per_pad)])
        return g_acc

    def trip_task(i0all, i1all, ikall, sign):
        nch = PT6 // GT6

        def chunk(ch, t_acc):
            di = pltpu.async_copy(zsh.at[i0all.at[ch]], bufi, sem)
            dj_ = pltpu.async_copy(zsh.at[i1all.at[ch]], bufj, sem)
            dk_ = pltpu.async_copy(zsh.at[ikall.at[ch]], bufk, sem)
            di.wait()
            dj_.wait()
            dk_.wait()

            def group(g, t_in):
                rows = iota + g * 16
                dj = jnp.zeros((16,), jnp.float32)
                dk = jnp.zeros((16,), jnp.float32)
                for dd in range(H):
                    cols = jnp.full((16,), dd, jnp.int32)
                    zi = plsc.load_gather(bufi, [rows, cols])
                    zj = plsc.load_gather(bufj, [rows, cols])
                    zk = plsc.load_gather(bufk, [rows, cols])
                    t1 = zi - zj
                    t2 = zi - zk
                    dj = dj + t1 * t1
                    dk = dk + t2 * t2
                out = (dj - dk) if sign > 0 else (dk - dj)
                out = jnp.maximum(out, 0.0)
                je = ch * GT6 + g * 16 + iota
                return t_in + jnp.where(je < E // NW, out, 0.0)

            return lax.fori_loop(0, GT6 // 16, group, t_acc)

        return lax.fori_loop(0, nch, chunk, jnp.zeros((16,), jnp.float32))

    nr, nrn = PT6 // GC6, PT6N // GC6
    pltpu.sync_copy(pe0.at[pl.ds(wid * nr, nr)], ia)
    pltpu.sync_copy(pe1.at[pl.ds(wid * nr, nr)], ib)
    pltpu.sync_copy(kpp.at[pl.ds(wid * nr, nr)], ik)
    pltpu.sync_copy(no0.at[pl.ds(wid * nrn, nrn)], ioa)
    pltpu.sync_copy(no1.at[pl.ds(wid * nrn, nrn)], iob)
    gp = nll_task(ia, ib, 0, PT6, 2500, S_POS)
    tp = trip_task(ia, ib, ik, +1)
    g0 = nll_task(ioa, iob, 2, PT6N, 5000, S_NONE)
    pltpu.sync_copy(ne0.at[pl.ds(wid * nr, nr)], ia)
    pltpu.sync_copy(ne1.at[pl.ds(wid * nr, nr)], ib)
    pltpu.sync_copy(knp.at[pl.ds(wid * nr, nr)], ik)
    gn = nll_task(ia, ib, 1, PT6, 2500, S_NEG)
    tn = trip_task(ia, ib, ik, -1)

    pbuf[pl.ds(0, 16)] = gp
    pbuf[pl.ds(16, 16)] = gn
    pbuf[pl.ds(32, 16)] = g0
    pbuf[pl.ds(48, 16)] = tp
    pbuf[pl.ds(64, 16)] = tn
    zero16 = jnp.zeros((16,), jnp.float32)
    pbuf[pl.ds(80, 16)] = zero16
    pbuf[pl.ds(96, 16)] = zero16
    pbuf[pl.ds(112, 16)] = zero16
    pltpu.sync_copy(pbuf, p_out.at[pl.ds(wid * 128, 128)])


def _k7_body(s_ref, p_ref, out_ref):
    ls = jnp.log(s_ref[...])
    rp = NW * PT6 // 128
    slp = jnp.sum(ls[0:rp])
    sln = jnp.sum(ls[rp:2 * rp])
    sl0 = jnp.sum(ls[2 * rp:])
    q = jnp.sum(p_ref[...], axis=1)
    gp, gn, g0, tp, tn = q[0], q[1], q[2], q[3], q[4]
    fe = jnp.float32(E)
    nll = ((gp + slp) / fe + (gn + sln) / fe + (g0 + sl0) / (2 * fe)) / 3.0
    loss = nll + LAMB * (tp / fe + tn / fe)
    out_ref[0, 0] = loss


def _k7(s_flat, p_flat):
    s2 = s_flat.reshape(S_TOT // 128, 128)
    p2 = p_flat.reshape(NW, 8, 16).transpose(1, 0, 2).reshape(8, NW * 16)
    return pl.pallas_call(
        _k7_body,
        out_shape=jax.ShapeDtypeStruct((1, 1), jnp.float32),
        out_specs=pl.BlockSpec(memory_space=pltpu.SMEM),
    )(s2, p2)


def kernel(positive_edges, negative_edges, target, X,
           Wp1, bp1, Wn1, bn1, Wp2, bp2, Wn2, bn2, Wd, bd,
           none_edges, k_pos, k_neg):
    del target
    pe0, pe1 = positive_edges[0], positive_edges[1]
    ne0, ne1 = negative_edges[0], negative_edges[1]

    w1cat = jnp.concatenate([Wp1[:D], Wn1[:D], Wp1[D:], Wn1[D:]], axis=1)
    b1cat = jnp.concatenate([bp1, bn1]).reshape(1, H)
    zblk = jnp.zeros((H2, H2), jnp.float32)
    wmp = jnp.block([[Wp2[0:H2], zblk], [zblk, Wn2[0:H2]]])
    wmn = jnp.block([[zblk, Wn2[H2:2 * H2]], [Wp2[H2:2 * H2], zblk]])
    wz = jnp.block([[Wp2[2 * H2:3 * H2], zblk], [zblk, Wn2[2 * H2:3 * H2]]])
    b2 = jnp.concatenate([bp2, bn2]).reshape(1, H)
    wd16 = jnp.zeros((H, 16), jnp.float32).at[:, 0:3].set(Wd[:H]).at[:, 4:7].set(Wd[H:])
    bd16 = jnp.zeros((16,), jnp.float32).at[0:3].set(bd).reshape(1, 16)

    per2 = E // NS
    src4 = jnp.concatenate([_pad_idx(pe0, per2, PT2, 0),
                            _pad_idx(ne0, per2, PT2, 0)]).reshape(-1, GC2)
    dst24 = jnp.concatenate([_pad_idx(pe1, per2, PT2, N),
                             _pad_idx(ne1, per2, PT2, N)]).reshape(-1, GC2)
    per6 = E // NW
    pe0p = _pad_idx(pe0, per6, PT6, 0).reshape(-1, GC6)
    pe1p = _pad_idx(pe1, per6, PT6, 0).reshape(-1, GC6)
    ne0p = _pad_idx(ne0, per6, PT6, 0).reshape(-1, GC6)
    ne1p = _pad_idx(ne1, per6, PT6, 0).reshape(-1, GC6)
    no0p = _pad_idx(none_edges[0], 2 * per6, PT6N, 0).reshape(-1, GC6)
    no1p = _pad_idx(none_edges[1], 2 * per6, PT6N, 0).reshape(-1, GC6)
    kpp = _pad_idx(k_pos, per6, PT6, 0).reshape(-1, GC6)
    knp = _pad_idx(k_neg, per6, PT6, 0).reshape(-1, GC6)

    z48 = jnp.zeros((N16, W1), jnp.float32)
    z64 = jnp.zeros((N16, H), jnp.float32)

    t1, xd = _k1(X, w1cat, b1cat)
    kacc = _segsum48(t1, src4, dst24, z48)
    z1 = _k3(kacc, xd)
    macc = _segsum64(z1, src4, dst24, z64)
    z, ab = _k5(macc, kacc, z1, wmp, wmn, wz, b2, wd16, bd16)
    s_flat, p_flat = _k6(ab, z, pe0p, pe1p, ne0p, ne1p, no0p, no1p, kpp, knp)
    loss = _k7(s_flat, p_flat)[0, 0]
    return (loss, z)
